# baseline (speedup 1.0000x reference)
import functools

import jax
import jax.numpy as jnp
from jax import lax
from jax.experimental import pallas as pl
from jax.experimental.pallas import tpu as pltpu
from jax.experimental.pallas import tpu_sc as plsc

N = 10000
E = 320000
D = 128
ALPHA = 0.5

NC = 2
NS = 16
NW = NC * NS
LANES = 16
VPD = D // LANES

EPW = E // NW
K = 40
K2 = 2 * K
NCHUNK = EPW // K
TOTCHUNK = E // K
RPS = 624
ZR = 48
REM = N - NS * RPS


def _pre_body(z_ref, x_ref, nf_ref, w_ref, h_ref):
    s = z_ref[...] + x_ref[...]
    h = lax.dot_general(s, w_ref[...], (((1,), (1,)), ((), ())),
                        preferred_element_type=jnp.float32)
    h_ref[...] = nf_ref[...] * h


def _post_body(z_ref, nf_ref, a_ref, w_ref, o_ref):
    s = nf_ref[...] * (a_ref[0] + a_ref[1])
    m = lax.dot_general(s, w_ref[...], (((1,), (0,)), ((), ())),
                        preferred_element_type=jnp.float32)
    o_ref[...] = (1.0 - ALPHA) * z_ref[...] - ALPHA * m


def _sc_body(h_hbm, ecomb_hbm, out_hbm,
             combA, combB, combSA, combSB,
             bufA, bufB, valsA, valsB, zbuf, acc,
             semGA, semGB, semIA, semIB, semSA, semSB):
    cid = lax.axis_index("c")
    sid = lax.axis_index("s")
    wid = sid * NC + cid

    @pl.loop(0, ZR)
    def _zero_fill(i):
        for j in range(VPD):
            zbuf[i, pl.ds(j * LANES, LANES)] = jnp.zeros((LANES,), jnp.float32)

    for b in range(RPS // ZR):
        pltpu.sync_copy(zbuf, acc.at[pl.ds(sid * RPS + b * ZR, ZR)])

    @pl.when(sid == NS - 1)
    def _zero_tail():
        pltpu.sync_copy(zbuf.at[pl.ds(0, REM)], acc.at[pl.ds(NS * RPS, REM)])

    plsc.subcore_barrier()

    gbase = wid * NCHUNK

    def load_idx(c, comb, sem):
        off = pl.multiple_of((gbase + c) * K2, 8)
        pltpu.async_copy(ecomb_hbm.at[pl.ds(off, K2)], comb, sem)

    def wait_idx(comb, sem):
        pltpu.make_async_copy(ecomb_hbm.at[pl.ds(0, K2)], comb, sem).wait()

    def start_gather(comb, buf, sem):
        pltpu.async_copy(h_hbm.at[comb], buf, sem)

    def wait_gather(comb, buf, sem):
        pltpu.make_async_copy(h_hbm.at[comb], buf, sem).wait()

    def compute(buf, vals):
        @plsc.parallel_loop(0, K, unroll=8)
        def _compute(i):
            for j in range(VPD):
                sl = pl.ds(j * LANES, LANES)
                v = buf[i, sl] - buf[K + i, sl]
                m = jnp.maximum(v, 0.0)
                vals[i, sl] = m
                vals[K + i, sl] = -m

    def start_scatter(combS, vals, sem):
        pltpu.async_copy(vals, acc.at[combS], sem, add=True)

    def wait_scatter(combS, vals, sem):
        pltpu.make_async_copy(vals, acc.at[combS], sem).wait()

    def copy_idx(src, dst):
        for o in range(0, K2, LANES):
            dst[pl.ds(o, LANES)] = src[pl.ds(o, LANES)]

    def phase(c, comb, combS, buf, vals, semG, semI, semS, pre):
        wait_gather(comb, buf, semG)

        @pl.when(c > 1)
        def _drain():
            wait_scatter(combS, vals, semS)

        copy_idx(comb, combS)

        @pl.when(pre)
        def _prefetch():
            load_idx(c + 2, comb, semI)

        compute(buf, vals)
        start_scatter(combS, vals, semS)

        @pl.when(pre)
        def _launch():
            wait_idx(comb, semI)
            start_gather(comb, buf, semG)

    pltpu.sync_copy(ecomb_hbm.at[pl.ds(pl.multiple_of(gbase * K2, 8), K2)],
                    combA)
    pltpu.sync_copy(ecomb_hbm.at[pl.ds(pl.multiple_of((gbase + 1) * K2, 8),
                                       K2)], combB)
    start_gather(combA, bufA, semGA)
    start_gather(combB, bufB, semGB)

    @pl.loop(0, NCHUNK, step=2)
    def _pair(c):
        more = c + 2 < NCHUNK
        phase(c, combA, combSA, bufA, valsA, semGA, semIA, semSA, more)
        phase(c + 1, combB, combSB, bufB, valsB, semGB, semIB, semSB, more)

    wait_scatter(combSA, valsA, semSA)
    wait_scatter(combSB, valsB, semSB)

    plsc.subcore_barrier()
    pltpu.sync_copy(acc.at[pl.ds(sid * RPS, RPS)],
                    out_hbm.at[cid, pl.ds(sid * RPS, RPS)])

    @pl.when(sid == NS - 1)
    def _copy_tail():
        pltpu.sync_copy(acc.at[pl.ds(NS * RPS, REM)],
                        out_hbm.at[cid, pl.ds(NS * RPS, REM)])


@functools.partial(
    pl.kernel,
    out_type=jax.ShapeDtypeStruct((NC, N, D), jnp.float32),
    mesh=plsc.VectorSubcoreMesh(core_axis_name="c", subcore_axis_name="s"),
    scratch_types=(
        [pltpu.VMEM((K2,), jnp.int32)] * 4
        + [pltpu.VMEM((K2, D), jnp.float32)] * 4
        + [pltpu.VMEM((ZR, D), jnp.float32),
           pltpu.VMEM_SHARED((N, D), jnp.float32)]
        + [pltpu.SemaphoreType.DMA] * 6
    ),
)
def _sc_edge_kernel(h_hbm, ecomb_hbm, out_hbm, *rest):
    _sc_body(h_hbm, ecomb_hbm, out_hbm, *rest)


def kernel(z, x, edge_index, norm_factor, batch, W):
    del batch

    ei = edge_index.reshape(2, TOTCHUNK, K)
    ecomb = jnp.concatenate([ei[0], ei[1]], axis=1).reshape(-1)

    BN = 2000
    h = pl.pallas_call(
        _pre_body,
        grid=(N // BN,),
        in_specs=[
            pl.BlockSpec((BN, D), lambda i: (i, 0)),
            pl.BlockSpec((BN, D), lambda i: (i, 0)),
            pl.BlockSpec((BN, 1), lambda i: (i, 0)),
            pl.BlockSpec((D, D), lambda i: (0, 0)),
        ],
        out_specs=pl.BlockSpec((BN, D), lambda i: (i, 0)),
        out_shape=jax.ShapeDtypeStruct((N, D), jnp.float32),
    )(z, x, norm_factor, W)

    a = _sc_edge_kernel(h, ecomb)

    out = pl.pallas_call(
        _post_body,
        grid=(N // BN,),
        in_specs=[
            pl.BlockSpec((BN, D), lambda i: (i, 0)),
            pl.BlockSpec((BN, 1), lambda i: (i, 0)),
            pl.BlockSpec((NC, BN, D), lambda i: (0, i, 0)),
            pl.BlockSpec((D, D), lambda i: (0, 0)),
        ],
        out_specs=pl.BlockSpec((BN, D), lambda i: (i, 0)),
        out_shape=jax.ShapeDtypeStruct((N, D), jnp.float32),
    )(z, norm_factor, a, W)

    return out
